# baseline (speedup 1.0000x reference)
import jax
import jax.numpy as jnp
from jax import lax
from jax.experimental import pallas as pl
from jax.experimental.pallas import tpu as pltpu

ROWS = 1_000_000
COLS = 32
BR = 2000
T = ROWS // BR
NB = 4
OUT = T // NB


def _tc_body(x_ref, o_ref, *rest):
    bufs = rest[:NB]
    sin = rest[NB:2 * NB]
    sout = rest[2 * NB:]

    def base(g):
        return pl.multiple_of(jnp.minimum(g, T - 1) * BR, 8)

    def start_in(g, b):
        pltpu.make_async_copy(x_ref.at[pl.ds(base(g), BR)], bufs[b], sin[b]).start()

    for b in range(NB):
        start_in(jnp.int32(b), b)

    def outer(o, carry):
        g0 = o * NB
        for b in range(NB):
            g = g0 + b
            pltpu.make_async_copy(x_ref.at[pl.ds(base(g), BR)], bufs[b], sin[b]).wait()
            pltpu.make_async_copy(bufs[b], o_ref.at[pl.ds(base(g), BR)], sout[b]).start()
        for b in range(NB):
            g = g0 + b
            pltpu.make_async_copy(bufs[b], o_ref.at[pl.ds(base(g), BR)], sout[b]).wait()
            start_in(g + NB, b)
        return carry

    lax.fori_loop(0, OUT, outer, 0)
    for b in range(NB):
        pltpu.make_async_copy(x_ref.at[pl.ds(base(jnp.int32(T - 1)), BR)], bufs[b], sin[b]).wait()


@jax.jit
def kernel(x):
    return pl.pallas_call(
        _tc_body,
        in_specs=[pl.BlockSpec(memory_space=pl.ANY)],
        out_specs=pl.BlockSpec(memory_space=pl.ANY),
        out_shape=jax.ShapeDtypeStruct((ROWS, COLS), jnp.float32),
        scratch_shapes=(
            [pltpu.VMEM((BR, COLS), jnp.float32) for _ in range(NB)]
            + [pltpu.SemaphoreType.DMA for _ in range(2 * NB)]
        ),
    )(x)
